# Initial kernel scaffold; baseline (speedup 1.0000x reference)
import jax
import jax.numpy as jnp
from jax.experimental import pallas as pl


def kernel(nonblank, real_atoms, mol_index, sigma, pair_dist, pair_first, pair_second, pair_coord, q0):
    raise NotImplementedError("write your pallas kernel here")


# speedup vs baseline: 1.1674x; 1.1674x over previous
import math

import jax
import jax.numpy as jnp
from jax.experimental import pallas as pl

PI = math.pi
E2 = 14.399645478425668
RC = 15.0
ALPHA = 0.2
N_MOL = 256
N_ATOM = 96
N_REAL = N_MOL * N_ATOM
N_PAIRS = 786432

ROWS = 6144
COLS = 128
BLK = 512


def _erfc_pos(x):
    t = 1.0 / (1.0 + 0.3275911 * x)
    poly = t * (0.254829592 + t * (-0.284496736 + t * (1.421413741 + t * (-1.453152027 + t * 1.061405429))))
    return poly * jnp.exp(-(x * x))


def _pair_math(d_ref, s12_ref, qq_ref, q2_ref, cx_ref, cy_ref, cz_ref,
               e_ref, bx_ref, by_ref, bz_ref, *a_refs):
    d = d_ref[...]
    d2 = d * d
    s12 = s12_ref[...]
    f_ij = 1.0 - _erfc_pos(d * jax.lax.rsqrt(s12))
    damp = _erfc_pos(ALPHA * d)
    damp_rc = math.erfc(ALPHA * RC)
    d_damp_rc = (-2.0 / math.sqrt(PI) * ALPHA) * jnp.exp(-(RC * RC) * d2)
    c0 = d > RC
    r1 = 1.0 / d
    r2 = r1 * r1
    dmrc = d - RC
    Uqqs = f_ij * (damp * r1 - damp_rc / RC - dmrc * (d_damp_rc / RC - damp_rc / RC**2))
    Uqps = f_ij * (damp * r2 - damp_rc / RC**2 - dmrc * (d_damp_rc / RC**2 - 2.0 * damp_rc / RC**3))
    Upps = f_ij * (damp * r1 * r2 - damp_rc / RC**3 - dmrc * (d_damp_rc / RC**3 - 3.0 * damp_rc / RC**4))
    zero = jnp.zeros_like(d)
    Uqqs = jnp.where(c0, zero, Uqqs)
    Uqps = jnp.where(c0, zero, Uqps)
    Upps = jnp.where(c0, zero, Upps)
    e_ref[...] = qq_ref[...] * Uqqs
    bf = q2_ref[...] * Uqps * r1
    cx = cx_ref[...]
    cy = cy_ref[...]
    cz = cz_ref[...]
    bx_ref[...] = bf * cx
    by_ref[...] = bf * cy
    bz_ref[...] = bf * cz
    w = E2 * Upps
    g = 3.0 * w * r2
    cc = (cx, cy, cz)
    for i in range(3):
        for j in range(3):
            v = g * cc[i] * cc[j]
            if i == j:
                v = v - w
            a_refs[i * 3 + j][...] = v


def kernel(nonblank, real_atoms, mol_index, sigma, pair_dist, pair_first, pair_second, pair_coord, q0):
    dtype = pair_dist.dtype
    sigma2 = sigma * sigma
    q = q0.reshape(-1)
    s12 = sigma2[pair_first] + sigma2[pair_second]
    qq = q[pair_first] * q[pair_second]
    q2g = q[pair_second]

    def r2d(x):
        return x.reshape(ROWS, COLS)

    ins = [r2d(pair_dist), r2d(s12), r2d(qq), r2d(q2g),
           r2d(pair_coord[:, 0]), r2d(pair_coord[:, 1]), r2d(pair_coord[:, 2])]
    spec = pl.BlockSpec((BLK, COLS), lambda i: (i, 0))
    outs = pl.pallas_call(
        _pair_math,
        grid=(ROWS // BLK,),
        in_specs=[spec] * 7,
        out_specs=[spec] * 13,
        out_shape=[jax.ShapeDtypeStruct((ROWS, COLS), dtype)] * 13,
    )(*ins)
    e = outs[0].reshape(-1)
    bvec = jnp.stack([o.reshape(-1) for o in outs[1:4]], axis=-1)
    app0 = jnp.stack([o.reshape(-1) for o in outs[4:]], axis=-1).reshape(N_PAIRS, 3, 3)

    Eqq0 = jnp.zeros((N_MOL,), dtype).at[mol_index[pair_first]].add(e)
    Eqq = 0.5 * E2 * Eqq0[:, None]
    bq1 = jnp.zeros((N_REAL, 3), dtype).at[pair_first].add(bvec)
    bq = (bq1 * E2).reshape(N_MOL, N_ATOM * 3)
    pair_mask = pair_first * N_ATOM + jnp.remainder(pair_second, N_ATOM)
    App1 = jnp.zeros((N_MOL * N_ATOM * N_ATOM, 3, 3), dtype).at[pair_mask].add(app0)
    App = App1.reshape(N_MOL, N_ATOM, N_ATOM, 3, 3).transpose(0, 1, 3, 2, 4).reshape(N_MOL, N_ATOM * 3, N_ATOM * 3)
    return (App, bq, Eqq)
